# baseline (speedup 1.0000x reference)
import math

import jax
import jax.numpy as jnp
from jax import lax
from jax.experimental import pallas as pl
from jax.experimental.pallas import tpu as pltpu
from jax.experimental.pallas import tpu_sc as plsc

NUM_EMB = 1000000
DIM = 64
BATCH = 4096
SEQ = 200

NC = 2
NS = 16
NW = NC * NS
B_PER = BATCH // NW

NBUF = 4
LOOKAHEAD = 2
SPLITS = ((0, 104), (104, 96))


def _pe_table():
    position = jnp.arange(0.0, SEQ)[:, None]
    div_term = jnp.exp(
        jnp.arange(0.0, DIM, 2) * -(math.log(10000.0) / DIM))
    tmp = position * div_term
    pe = jnp.zeros((SEQ, DIM), dtype=jnp.float32)
    pe = pe.at[:, 0::2].set(jnp.sin(tmp))
    pe = pe.at[:, 1::2].set(jnp.cos(tmp))
    return pe


def _body(table_hbm, inp_hbm, pe_hbm, out_hbm, idx_v, pe_v, gbuf, *sems):
    gsem = sems[:NBUF]
    ssem = sems[NBUF:]
    wid = lax.axis_index("c") * NS + lax.axis_index("s")
    base = wid * B_PER

    pltpu.sync_copy(inp_hbm.at[pl.ds(base, B_PER)], idx_v)
    pltpu.sync_copy(pe_hbm, pe_v)

    def issue_gather(b, p):
        for off, n in SPLITS:
            pltpu.async_copy(
                table_hbm.at[idx_v.at[b, pl.ds(off, n)]],
                gbuf.at[p, pl.ds(off, n)],
                gsem[p])

    def wait_chunk(sem, p):
        pltpu.make_async_copy(out_hbm.at[0], gbuf.at[p], sem).wait()

    for p in range(LOOKAHEAD):
        issue_gather(p, p)

    @pl.loop(0, B_PER // NBUF)
    def _sstep(ss):
        for p in range(NBUF):
            b = ss * NBUF + p
            wait_chunk(gsem[p], p)

            @pl.loop(0, SEQ)
            def _add(r):
                for c in range(DIM // 16):
                    plsc.addupdate(
                        gbuf.at[p, r, pl.ds(c * 16, 16)],
                        pe_v[r, pl.ds(c * 16, 16)])

            pltpu.async_copy(gbuf.at[p], out_hbm.at[base + b], ssem[p])

            g = b + LOOKAHEAD
            p2 = (p + LOOKAHEAD) % NBUF

            @pl.when(jnp.logical_and(g < B_PER, g >= NBUF))
            def _():
                wait_chunk(ssem[p2], p2)

            @pl.when(g < B_PER)
            def _():
                issue_gather(g, p2)

    for p in range(NBUF):
        wait_chunk(ssem[p], p)


def kernel(inputs, table):
    pe = _pe_table()
    idx = inputs.astype(jnp.int32)
    mesh = plsc.VectorSubcoreMesh(core_axis_name="c", subcore_axis_name="s")
    kfn = pl.kernel(
        _body,
        out_type=jax.ShapeDtypeStruct((BATCH, SEQ, DIM), jnp.float32),
        mesh=mesh,
        scratch_types=(
            [pltpu.VMEM((B_PER, SEQ), jnp.int32),
             pltpu.VMEM((SEQ, DIM), jnp.float32),
             pltpu.VMEM((NBUF, SEQ, DIM), jnp.float32)]
            + [pltpu.SemaphoreType.DMA] * (2 * NBUF)),
        compiler_params=pltpu.CompilerParams(use_tc_tiling_on_sc=False),
    )
    return kfn(table, idx, pe)
